# Initial kernel scaffold; baseline (speedup 1.0000x reference)
import jax
import jax.numpy as jnp
from jax.experimental import pallas as pl


def kernel(embeddings, sp_seg, edge_index, W):
    raise NotImplementedError("write your pallas kernel here")


# speedup vs baseline: 6.3708x; 6.3708x over previous
import functools

import jax
import jax.numpy as jnp
from jax import lax
from jax.experimental import pallas as pl
from jax.experimental.pallas import tpu as pltpu
from jax.experimental.pallas import tpu_sc as plsc

F32 = jnp.float32
I32 = jnp.int32

BS, D, H, Wd = 16, 32, 128, 128
SP_PER_IMG = 1024
NUM_SEG = BS * SP_PER_IMG + BS
N_TOK = BS * H * Wd
E = 262144

NC, NS = 2, 16
NW = NC * NS
SLICE = 1040
SEG_PAD = NS * SLICE
CH = 1024
RPC = 128
CALLS = CH // RPC
TPW = N_TOK // NW
NCH = TPW // CH
CW = 16
ZR = 520


def _k1_body(x_ref, out_ref):
    x = x_ref[...]
    m = jnp.max(x, axis=1, keepdims=True)
    acc = m
    for k in (1, 2, 4, 8):
        shifted = jnp.concatenate([jnp.zeros((k, 1), I32), acc[:BS - k]], axis=0)
        acc = acc + shifted
    off_i = acc - m + lax.broadcasted_iota(I32, (BS, 1), 0)
    out_ref[...] = x + off_i


def _global_ids(sp_seg):
    x = sp_seg.reshape(BS, H * Wd)
    return pl.pallas_call(
        _k1_body,
        out_shape=jax.ShapeDtypeStruct((BS, H * Wd), I32),
    )(x)


def _fill_rows(ref, nrows, ncols, vec):
    def body(i, carry):
        for j0 in range(ncols // 16):
            ref[i, pl.ds(j0 * 16, 16)] = vec
        return carry
    lax.fori_loop(0, nrows, body, 0)


def _seg_accum_body(rows_hbm, ids_hbm, sums_out, cnt_out,
                    tok_v, idx_v, ones_v, zb_v, sums_sh, cnt_sh):
    c = lax.axis_index("c")
    s = lax.axis_index("s")
    wid = c * NS + s
    zero16 = jnp.zeros((16,), F32)
    one16 = jnp.ones((16,), F32)
    _fill_rows(ones_v, RPC, CW, one16)
    _fill_rows(zb_v, ZR, CW, zero16)
    _fill_rows(tok_v, CH, D, zero16)
    pltpu.sync_copy(tok_v, sums_sh.at[pl.ds(s * SLICE, CH)])
    pltpu.sync_copy(tok_v.at[pl.ds(0, SLICE - CH)],
                    sums_sh.at[pl.ds(s * SLICE + CH, SLICE - CH)])
    pltpu.sync_copy(zb_v, cnt_sh.at[pl.ds(s * SLICE, ZR)])
    pltpu.sync_copy(zb_v, cnt_sh.at[pl.ds(s * SLICE + ZR, SLICE - ZR)])
    plsc.subcore_barrier()
    for ch in range(NCH):
        base = wid * TPW + ch * CH
        pltpu.sync_copy(rows_hbm.at[pl.ds(base, CH)], tok_v)
        pltpu.sync_copy(ids_hbm.at[pl.ds(wid * (TPW // RPC) + ch * CALLS, CALLS)],
                        idx_v)
        for j in range(CALLS):
            pltpu.sync_copy(tok_v.at[pl.ds(j * RPC, RPC)],
                            sums_sh.at[idx_v.at[j]], add=True)
            pltpu.sync_copy(ones_v, cnt_sh.at[idx_v.at[j]], add=True)
    plsc.subcore_barrier()
    pltpu.sync_copy(sums_sh.at[pl.ds(s * SLICE, SLICE)],
                    sums_out.at[c, pl.ds(s * SLICE, SLICE)])
    pltpu.sync_copy(cnt_sh.at[pl.ds(s * SLICE, SLICE)],
                    cnt_out.at[c, pl.ds(s * SLICE, SLICE)])


_seg_accum = functools.partial(
    pl.kernel,
    out_type=(jax.ShapeDtypeStruct((NC, SEG_PAD, D), F32),
              jax.ShapeDtypeStruct((NC, SEG_PAD, CW), F32)),
    mesh=plsc.VectorSubcoreMesh(core_axis_name="c", subcore_axis_name="s", num_cores=NC, num_subcores=NS),
    compiler_params=pltpu.CompilerParams(use_tc_tiling_on_sc=False),
    scratch_types=[
        pltpu.VMEM((CH, D), F32),
        pltpu.VMEM((CALLS, RPC), I32),
        pltpu.VMEM((RPC, CW), F32),
        pltpu.VMEM((ZR, CW), F32),
        pltpu.VMEM_SHARED((SEG_PAD, D), F32),
        pltpu.VMEM_SHARED((SEG_PAD, CW), F32),
    ],
)(_seg_accum_body)


def _edge_accum_body(table_hbm, src_hbm, dst_hbm, sums_out, cnt_out,
                     tok_v, sidx_v, didx_v, ones_v, zb_v, sums_sh, cnt_sh,
                     sem):
    c = lax.axis_index("c")
    s = lax.axis_index("s")
    wid = c * NS + s
    zero16 = jnp.zeros((16,), F32)
    one16 = jnp.ones((16,), F32)
    _fill_rows(ones_v, RPC, CW, one16)
    _fill_rows(zb_v, ZR, CW, zero16)
    _fill_rows(tok_v, CH, D, zero16)
    pltpu.sync_copy(tok_v, sums_sh.at[pl.ds(s * SLICE, CH)])
    pltpu.sync_copy(tok_v.at[pl.ds(0, SLICE - CH)],
                    sums_sh.at[pl.ds(s * SLICE + CH, SLICE - CH)])
    pltpu.sync_copy(zb_v, cnt_sh.at[pl.ds(s * SLICE, ZR)])
    pltpu.sync_copy(zb_v, cnt_sh.at[pl.ds(s * SLICE + ZR, SLICE - ZR)])
    plsc.subcore_barrier()
    for ch in range(NCH):
        row0 = wid * (TPW // RPC) + ch * CALLS
        pltpu.sync_copy(src_hbm.at[pl.ds(row0, CALLS)], sidx_v)
        pltpu.sync_copy(dst_hbm.at[pl.ds(row0, CALLS)], didx_v)
        for j in range(CALLS):
            pltpu.async_copy(table_hbm.at[sidx_v.at[j]],
                             tok_v.at[pl.ds(j * RPC, RPC)], sem).wait()
        for j in range(CALLS):
            pltpu.sync_copy(tok_v.at[pl.ds(j * RPC, RPC)],
                            sums_sh.at[didx_v.at[j]], add=True)
            pltpu.sync_copy(ones_v, cnt_sh.at[didx_v.at[j]], add=True)
    plsc.subcore_barrier()
    pltpu.sync_copy(sums_sh.at[pl.ds(s * SLICE, SLICE)],
                    sums_out.at[c, pl.ds(s * SLICE, SLICE)])
    pltpu.sync_copy(cnt_sh.at[pl.ds(s * SLICE, SLICE)],
                    cnt_out.at[c, pl.ds(s * SLICE, SLICE)])


_edge_accum = functools.partial(
    pl.kernel,
    out_type=(jax.ShapeDtypeStruct((NC, SEG_PAD, D), F32),
              jax.ShapeDtypeStruct((NC, SEG_PAD, CW), F32)),
    mesh=plsc.VectorSubcoreMesh(core_axis_name="c", subcore_axis_name="s", num_cores=NC, num_subcores=NS),
    compiler_params=pltpu.CompilerParams(use_tc_tiling_on_sc=False),
    scratch_types=[
        pltpu.VMEM((CH, D), F32),
        pltpu.VMEM((CALLS, RPC), I32),
        pltpu.VMEM((CALLS, RPC), I32),
        pltpu.VMEM((RPC, CW), F32),
        pltpu.VMEM((ZR, CW), F32),
        pltpu.VMEM_SHARED((SEG_PAD, D), F32),
        pltpu.VMEM_SHARED((SEG_PAD, CW), F32),
        pltpu.SemaphoreType.DMA,
    ],
)(_edge_accum_body)


_RB = SEG_PAD // 8


def _k3_body(sums_ref, cnt_ref, out_ref):
    ssum = sums_ref[0] + sums_ref[1]
    cnt = cnt_ref[0, :, 0:1] + cnt_ref[1, :, 0:1]
    out_ref[...] = ssum / jnp.maximum(cnt, 1.0)


def _seg_mean(sums, cnts):
    return pl.pallas_call(
        _k3_body,
        grid=(SEG_PAD // _RB,),
        in_specs=[
            pl.BlockSpec((NC, _RB, D), lambda i: (0, i, 0)),
            pl.BlockSpec((NC, _RB, CW), lambda i: (0, i, 0)),
        ],
        out_specs=pl.BlockSpec((_RB, D), lambda i: (i, 0)),
        out_shape=jax.ShapeDtypeStruct((SEG_PAD, D), F32),
    )(sums, cnts)


def _k5_body(f_ref, a_ref, c_ref, w_ref, out_ref):
    agg = (a_ref[0] + a_ref[1]) / jnp.maximum(
        c_ref[0, :, 0:1] + c_ref[1, :, 0:1], 1.0)
    t = f_ref[...] + agg
    o = jnp.dot(t, w_ref[...], preferred_element_type=F32)
    n = jnp.sqrt(jnp.sum(o * o, axis=1, keepdims=True)) + 1e-12
    out_ref[...] = o / n


def _head(features_sp, agg_sums, agg_cnts, W):
    return pl.pallas_call(
        _k5_body,
        grid=(SEG_PAD // _RB,),
        in_specs=[
            pl.BlockSpec((_RB, D), lambda i: (i, 0)),
            pl.BlockSpec((NC, _RB, D), lambda i: (0, i, 0)),
            pl.BlockSpec((NC, _RB, CW), lambda i: (0, i, 0)),
            pl.BlockSpec((D, D), lambda i: (0, 0)),
        ],
        out_specs=pl.BlockSpec((_RB, D), lambda i: (i, 0)),
        out_shape=jax.ShapeDtypeStruct((SEG_PAD, D), F32),
    )(features_sp, agg_sums, agg_cnts, W)


def kernel(embeddings, sp_seg, edge_index, W):
    emb = jnp.transpose(embeddings, (0, 2, 3, 1)).reshape(N_TOK, D)
    sp = _global_ids(sp_seg).reshape(N_TOK // RPC, RPC)
    sums, cnts = _seg_accum(emb, sp)
    features_sp = _seg_mean(sums, cnts)
    src = edge_index[0].reshape(E // RPC, RPC)
    dst = edge_index[1].reshape(E // RPC, RPC)
    agg_sums, agg_cnts = _edge_accum(features_sp, src, dst)
    out = _head(features_sp, agg_sums, agg_cnts, W)
    return out[:NUM_SEG]


# speedup vs baseline: 7.6631x; 1.2028x over previous
import functools

import jax
import jax.numpy as jnp
from jax import lax
from jax.experimental import pallas as pl
from jax.experimental.pallas import tpu as pltpu
from jax.experimental.pallas import tpu_sc as plsc

F32 = jnp.float32
I32 = jnp.int32

BS, D, H, Wd = 16, 32, 128, 128
SP_PER_IMG = 1024
NUM_SEG = BS * SP_PER_IMG + BS
HW = H * Wd
N_TOK = BS * HW
E = 262144

NC, NS = 2, 16
NW = NC * NS
SLICE = 1040
SEG_PAD = NS * SLICE
CH = 1024
RPC = 128
CALLS = CH // RPC
TPW = N_TOK // NW
NCH = TPW // CH
CW = 16
ZR = 520


def _k1_body(x_ref, out_ref):
    x = x_ref[...]
    m = jnp.max(jnp.max(x, axis=2, keepdims=True), axis=1, keepdims=True)
    acc = m
    for k in (1, 2, 4, 8):
        shifted = jnp.concatenate(
            [jnp.zeros((k, 1, 1), I32), acc[:BS - k]], axis=0)
        acc = acc + shifted
    off = acc - m + lax.broadcasted_iota(I32, (BS, 1, 1), 0)
    out_ref[...] = x + off


def _global_ids(sp_seg):
    return pl.pallas_call(
        _k1_body,
        out_shape=jax.ShapeDtypeStruct((BS, H, Wd), I32),
    )(sp_seg)


def _fill_rows(ref, nrows, ncols, vec):
    def body(i, carry):
        for j0 in range(ncols // 16):
            ref[i, pl.ds(j0 * 16, 16)] = vec
        return carry
    lax.fori_loop(0, nrows, body, 0)


def _zero_shared(s, tok_v, zb_v, sums_sh, cnt_sh):
    zero16 = jnp.zeros((16,), F32)
    _fill_rows(zb_v, ZR, CW, zero16)
    _fill_rows(tok_v, CH, D, zero16)
    pltpu.sync_copy(tok_v, sums_sh.at[pl.ds(s * SLICE, CH)])
    pltpu.sync_copy(tok_v.at[pl.ds(0, SLICE - CH)],
                    sums_sh.at[pl.ds(s * SLICE + CH, SLICE - CH)])
    pltpu.sync_copy(zb_v, cnt_sh.at[pl.ds(s * SLICE, ZR)])
    pltpu.sync_copy(zb_v, cnt_sh.at[pl.ds(s * SLICE + ZR, SLICE - ZR)])


def _writeback(c, s, sums_sh, cnt_sh, sums_out, cnt_out):
    pltpu.sync_copy(sums_sh.at[pl.ds(s * SLICE, SLICE)],
                    sums_out.at[c, pl.ds(s * SLICE, SLICE)])
    pltpu.sync_copy(cnt_sh.at[pl.ds(s * SLICE, SLICE)],
                    cnt_out.at[c, pl.ds(s * SLICE, SLICE)])


def _seg_accum_body(rows_hbm, ids_hbm, sums_out, cnt_out,
                    tokA, tokB, idxA, idxB, ones_v, zb_v, sums_sh, cnt_sh,
                    sem_in, sem_add):
    c = lax.axis_index("c")
    s = lax.axis_index("s")
    wid = c * NS + s
    b = wid // 2
    half = wid % 2
    _fill_rows(ones_v, RPC, CW, jnp.ones((16,), F32))
    _zero_shared(s, tokA, zb_v, sums_sh, cnt_sh)
    plsc.subcore_barrier()

    toks = [tokA, tokB]
    idxs = [idxA, idxB]

    def fire_in(ch):
        tok, idx = toks[ch % 2], idxs[ch % 2]
        hw0 = half * (HW // 2) + ch * CH
        r0 = half * 64 + ch * CALLS
        return [pltpu.async_copy(rows_hbm.at[b, pl.ds(hw0, CH)], tok, sem_in),
                pltpu.async_copy(ids_hbm.at[b, pl.ds(r0, CALLS)], idx, sem_in)]

    pend = fire_in(0)
    for ch in range(NCH):
        tok, idx = toks[ch % 2], idxs[ch % 2]
        for cp in pend:
            cp.wait()
        if ch + 1 < NCH:
            pend = fire_in(ch + 1)
        adds = []
        for j in range(CALLS):
            adds.append(pltpu.async_copy(
                tok.at[pl.ds(j * RPC, RPC)], sums_sh.at[idx.at[j]],
                sem_add, add=True))
            adds.append(pltpu.async_copy(
                ones_v, cnt_sh.at[idx.at[j]], sem_add, add=True))
        for cp in adds:
            cp.wait()

    plsc.subcore_barrier()
    _writeback(c, s, sums_sh, cnt_sh, sums_out, cnt_out)


_seg_accum = functools.partial(
    pl.kernel,
    out_type=(jax.ShapeDtypeStruct((NC, SEG_PAD, D), F32),
              jax.ShapeDtypeStruct((NC, SEG_PAD, CW), F32)),
    mesh=plsc.VectorSubcoreMesh(core_axis_name="c", subcore_axis_name="s",
                                num_cores=NC, num_subcores=NS),
    compiler_params=pltpu.CompilerParams(use_tc_tiling_on_sc=False),
    scratch_types=[
        pltpu.VMEM((CH, D), F32),
        pltpu.VMEM((CH, D), F32),
        pltpu.VMEM((CALLS, RPC), I32),
        pltpu.VMEM((CALLS, RPC), I32),
        pltpu.VMEM((RPC, CW), F32),
        pltpu.VMEM((ZR, CW), F32),
        pltpu.VMEM_SHARED((SEG_PAD, D), F32),
        pltpu.VMEM_SHARED((SEG_PAD, CW), F32),
        pltpu.SemaphoreType.DMA,
        pltpu.SemaphoreType.DMA,
    ],
)(_seg_accum_body)


def _edge_accum_body(table_hbm, edge_hbm, sums_out, cnt_out,
                     tokA, tokB, sidxA, sidxB, didxA, didxB, ones_v, zb_v,
                     sums_sh, cnt_sh, sem_in, sem_g, sem_add):
    c = lax.axis_index("c")
    s = lax.axis_index("s")
    wid = c * NS + s
    _fill_rows(ones_v, RPC, CW, jnp.ones((16,), F32))
    _zero_shared(s, tokA, zb_v, sums_sh, cnt_sh)
    plsc.subcore_barrier()

    toks = [tokA, tokB]
    sidxs = [sidxA, sidxB]
    didxs = [didxA, didxB]

    def fire_idx(ch):
        r0 = wid * (TPW // RPC) + ch * CALLS
        return [pltpu.async_copy(edge_hbm.at[0, pl.ds(r0, CALLS)],
                                 sidxs[ch % 2], sem_in),
                pltpu.async_copy(edge_hbm.at[1, pl.ds(r0, CALLS)],
                                 didxs[ch % 2], sem_in)]

    def fire_gathers(ch):
        tok, sidx = toks[ch % 2], sidxs[ch % 2]
        return [pltpu.async_copy(table_hbm.at[sidx.at[j]],
                                 tok.at[pl.ds(j * RPC, RPC)], sem_g)
                for j in range(CALLS)]

    pend_idx = fire_idx(0)
    for cp in pend_idx:
        cp.wait()
    pend_g = fire_gathers(0)
    pend_idx = fire_idx(1)

    for ch in range(NCH):
        tok, didx = toks[ch % 2], didxs[ch % 2]
        for cp in pend_g:
            cp.wait()
        adds = []
        for j in range(CALLS):
            adds.append(pltpu.async_copy(
                tok.at[pl.ds(j * RPC, RPC)], sums_sh.at[didx.at[j]],
                sem_add, add=True))
            adds.append(pltpu.async_copy(
                ones_v, cnt_sh.at[didx.at[j]], sem_add, add=True))
        if ch + 1 < NCH:
            for cp in pend_idx:
                cp.wait()
            pend_g = fire_gathers(ch + 1)
        for cp in adds:
            cp.wait()
        if ch + 2 < NCH:
            pend_idx = fire_idx(ch + 2)

    plsc.subcore_barrier()
    _writeback(c, s, sums_sh, cnt_sh, sums_out, cnt_out)


_edge_accum = functools.partial(
    pl.kernel,
    out_type=(jax.ShapeDtypeStruct((NC, SEG_PAD, D), F32),
              jax.ShapeDtypeStruct((NC, SEG_PAD, CW), F32)),
    mesh=plsc.VectorSubcoreMesh(core_axis_name="c", subcore_axis_name="s",
                                num_cores=NC, num_subcores=NS),
    compiler_params=pltpu.CompilerParams(use_tc_tiling_on_sc=False),
    scratch_types=[
        pltpu.VMEM((CH, D), F32),
        pltpu.VMEM((CH, D), F32),
        pltpu.VMEM((CALLS, RPC), I32),
        pltpu.VMEM((CALLS, RPC), I32),
        pltpu.VMEM((CALLS, RPC), I32),
        pltpu.VMEM((CALLS, RPC), I32),
        pltpu.VMEM((RPC, CW), F32),
        pltpu.VMEM((ZR, CW), F32),
        pltpu.VMEM_SHARED((SEG_PAD, D), F32),
        pltpu.VMEM_SHARED((SEG_PAD, CW), F32),
        pltpu.SemaphoreType.DMA,
        pltpu.SemaphoreType.DMA,
        pltpu.SemaphoreType.DMA,
    ],
)(_edge_accum_body)


_RB = SEG_PAD // 8


def _k3_body(sums_ref, cnt_ref, out_ref):
    ssum = sums_ref[0] + sums_ref[1]
    cnt = cnt_ref[0, :, 0:1] + cnt_ref[1, :, 0:1]
    out_ref[...] = ssum / jnp.maximum(cnt, 1.0)


def _seg_mean(sums, cnts):
    return pl.pallas_call(
        _k3_body,
        grid=(SEG_PAD // _RB,),
        in_specs=[
            pl.BlockSpec((NC, _RB, D), lambda i: (0, i, 0)),
            pl.BlockSpec((NC, _RB, CW), lambda i: (0, i, 0)),
        ],
        out_specs=pl.BlockSpec((_RB, D), lambda i: (i, 0)),
        out_shape=jax.ShapeDtypeStruct((SEG_PAD, D), F32),
    )(sums, cnts)


_RB5 = 400


def _k5_body(f_ref, a_ref, c_ref, w_ref, out_ref):
    agg = (a_ref[0] + a_ref[1]) / jnp.maximum(
        c_ref[0, :, 0:1] + c_ref[1, :, 0:1], 1.0)
    t = f_ref[...] + agg
    o = jnp.dot(t, w_ref[...], preferred_element_type=F32)
    n = jnp.sqrt(jnp.sum(o * o, axis=1, keepdims=True)) + 1e-12
    out_ref[...] = o / n


def _head(features_sp, agg_sums, agg_cnts, W):
    return pl.pallas_call(
        _k5_body,
        grid=(NUM_SEG // _RB5,),
        in_specs=[
            pl.BlockSpec((_RB5, D), lambda i: (i, 0)),
            pl.BlockSpec((NC, _RB5, D), lambda i: (0, i, 0)),
            pl.BlockSpec((NC, _RB5, CW), lambda i: (0, i, 0)),
            pl.BlockSpec((D, D), lambda i: (0, 0)),
        ],
        out_specs=pl.BlockSpec((_RB5, D), lambda i: (i, 0)),
        out_shape=jax.ShapeDtypeStruct((NUM_SEG, D), F32),
    )(features_sp, agg_sums, agg_cnts, W)


def kernel(embeddings, sp_seg, edge_index, W):
    emb3 = jnp.transpose(embeddings, (0, 2, 3, 1)).reshape(BS, HW, D)
    sp3 = _global_ids(sp_seg)
    sums, cnts = _seg_accum(emb3, sp3)
    features_sp = _seg_mean(sums, cnts)
    edge3 = edge_index.reshape(2, E // RPC, RPC)
    agg_sums, agg_cnts = _edge_accum(features_sp, edge3)
    return _head(features_sp, agg_sums, agg_cnts, W)


# speedup vs baseline: 8.9812x; 1.1720x over previous
import functools

import jax
import jax.numpy as jnp
from jax import lax
from jax.experimental import pallas as pl
from jax.experimental.pallas import tpu as pltpu
from jax.experimental.pallas import tpu_sc as plsc

F32 = jnp.float32
I32 = jnp.int32

BS, D, H, Wd = 16, 32, 128, 128
SP_PER_IMG = 1024
NUM_SEG = BS * SP_PER_IMG + BS
HW = H * Wd
N_TOK = BS * HW
E = 262144

NC, NS = 2, 16
NW = NC * NS
IMG_ROWS = 1040
SEG_PAD = BS * IMG_ROWS
AW = 40
GARBAGE_ROW = SEG_PAD - 1

KR = 2
CH2 = KR * Wd
NCH2 = (HW // 2) // CH2

CH4 = 512
RPC = 128
CALLS4 = CH4 // RPC
EPT = E // NW
NCH4 = EPT // CH4
CW = 16
ZR = 520


def _k1_body(x_ref, out_ref):
    x = x_ref[...]
    m = jnp.max(jnp.max(x, axis=2, keepdims=True), axis=1, keepdims=True)
    acc = m
    for k in (1, 2, 4, 8):
        shifted = jnp.concatenate(
            [jnp.zeros((k, 1, 1), I32), acc[:BS - k]], axis=0)
        acc = acc + shifted
    out_ref[...] = acc - m + lax.broadcasted_iota(I32, (BS, 1, 1), 0)


def _offsets(sp_seg):
    return pl.pallas_call(
        _k1_body,
        out_shape=jax.ShapeDtypeStruct((BS, 1, 1), I32),
    )(sp_seg)


_SC_PARAMS = pltpu.CompilerParams(use_tc_tiling_on_sc=False,
                                  needs_layout_passes=False)
_MESH = plsc.VectorSubcoreMesh(core_axis_name="c", subcore_axis_name="s",
                               num_cores=NC, num_subcores=NS)


def _fill_rows(ref, nrows, ncols, vec):
    def body(i, carry):
        for j0 in range(ncols // 16):
            ref[i, pl.ds(j0 * 16, 16)] = vec
        return carry
    lax.fori_loop(0, nrows, body, 0)


def _pool_body(emb_hbm, sp_hbm, table_out,
               pbA, pbB, idA, idB, acc_v, iota_v, dv_v, ob_v, comb_sh,
               sem_in):
    c = lax.axis_index("c")
    s = lax.axis_index("s")
    wid = c * NS + s
    b = wid // 2
    half = wid % 2
    h0_base = half * (H // 2)
    zero16 = jnp.zeros((16,), F32)
    one16 = jnp.ones((16,), F32)
    iota16 = lax.iota(I32, 16)

    _fill_rows(acc_v, SP_PER_IMG, AW, zero16)
    pltpu.sync_copy(acc_v.at[pl.ds(0, ZR)],
                    comb_sh.at[pl.ds(s * ZR, ZR)])
    cbase = (b % 8) * IMG_ROWS
    for j in range(8):
        for k in range(8):
            iota_v[j, pl.ds(k * 16, 16)] = cbase + j * 128 + k * 16 + iota16
    plsc.subcore_barrier()

    pbs, ids = [pbA, pbB], [idA, idB]

    def fire(ch, which):
        h0 = h0_base + ch * KR
        return [pltpu.async_copy(emb_hbm.at[b, :, pl.ds(h0, KR), :],
                                 pbs[which], sem_in),
                pltpu.async_copy(sp_hbm.at[b, pl.ds(h0, KR), :],
                                 ids[which], sem_in)]

    def compute(which):
        pb, idb = pbs[which], ids[which]

        def gbody(g, carry):
            r = g // 8
            col = (g % 8) * 16
            seg16 = idb[r, pl.ds(col, 16)]
            for d in range(D):
                v16 = pb[d, r, pl.ds(col, 16)]
                plsc.addupdate_scatter(acc_v, [seg16, jnp.full((16,), d, I32)],
                                       v16)
            plsc.addupdate_scatter(acc_v, [seg16, jnp.full((16,), D, I32)],
                                   one16)
            return carry
        lax.fori_loop(0, CH2 // 16, gbody, 0)

    pend = fire(0, 0)
    for p in range(NCH2 // 2 - 1):
        for cp in pend:
            cp.wait()
        pend = fire(2 * p + 1, 1)
        compute(0)
        for cp in pend:
            cp.wait()
        pend = fire(2 * p + 2, 0)
        compute(1)
    for cp in pend:
        cp.wait()
    pend = fire(NCH2 - 1, 1)
    compute(0)
    for cp in pend:
        cp.wait()
    compute(1)

    for j in range(8):
        pltpu.sync_copy(acc_v.at[pl.ds(j * 128, 128)],
                        comb_sh.at[iota_v.at[j]], add=True)
    plsc.subcore_barrier()

    i2 = s // 2
    half2 = s % 2
    src0 = i2 * IMG_ROWS + half2 * ZR
    pltpu.sync_copy(comb_sh.at[pl.ds(src0, ZR)], dv_v)

    def dbody(r, carry):
        v1 = dv_v[r, pl.ds(0, 16)]
        v2 = dv_v[r, pl.ds(16, 16)]
        tail = dv_v[r, pl.ds(24, 16)]
        cnt = jnp.maximum(jnp.full((16,), tail[8], F32), 1.0)
        ob_v[r, pl.ds(0, 16)] = v1 / cnt
        ob_v[r, pl.ds(16, 16)] = v2 / cnt
        return carry
    lax.fori_loop(0, ZR, dbody, 0)

    dst0 = (c * 8 + i2) * IMG_ROWS + half2 * ZR
    pltpu.sync_copy(ob_v, table_out.at[pl.ds(dst0, ZR)])


_pool = functools.partial(
    pl.kernel,
    out_type=jax.ShapeDtypeStruct((SEG_PAD, D), F32),
    mesh=_MESH,
    compiler_params=_SC_PARAMS,
    scratch_types=[
        pltpu.VMEM((D, KR, Wd), F32),
        pltpu.VMEM((D, KR, Wd), F32),
        pltpu.VMEM((KR, Wd), I32),
        pltpu.VMEM((KR, Wd), I32),
        pltpu.VMEM((SP_PER_IMG, AW), F32),
        pltpu.VMEM((8, 128), I32),
        pltpu.VMEM((ZR, AW), F32),
        pltpu.VMEM((ZR, D), F32),
        pltpu.VMEM_SHARED((8 * IMG_ROWS, AW), F32),
        pltpu.SemaphoreType.DMA,
    ],
)(_pool_body)


def _edge_body(table_hbm, edge_hbm, off_hbm, sums_out, cnt_out, fc_out,
               tokA, tokB, sidxA, sidxB, s2A, s2B, didxA, didxB,
               off_v, corr_v, perm_v, ones_v, zb_v,
               sums_sh, cnt_sh, perm_sh, sem_in, sem_g, sem_add):
    c = lax.axis_index("c")
    s = lax.axis_index("s")
    wid = c * NS + s
    zero16 = jnp.zeros((16,), F32)
    iota16 = lax.iota(I32, 16)
    _fill_rows(ones_v, RPC, CW, jnp.ones((16,), F32))
    _fill_rows(zb_v, ZR, CW, zero16)
    _fill_rows(tokA, CH4, D, zero16)
    pltpu.sync_copy(tokA, sums_sh.at[pl.ds(s * IMG_ROWS, CH4)])
    pltpu.sync_copy(tokA, sums_sh.at[pl.ds(s * IMG_ROWS + CH4, CH4)])
    pltpu.sync_copy(tokA.at[pl.ds(0, IMG_ROWS - 2 * CH4)],
                    sums_sh.at[pl.ds(s * IMG_ROWS + 2 * CH4,
                                     IMG_ROWS - 2 * CH4)])
    pltpu.sync_copy(zb_v, cnt_sh.at[pl.ds(s * IMG_ROWS, ZR)])
    pltpu.sync_copy(zb_v, cnt_sh.at[pl.ds(s * IMG_ROWS + ZR, IMG_ROWS - ZR)])

    pltpu.sync_copy(off_hbm, off_v)
    off16 = off_v[...]
    corr_v[...] = iota16 * IMG_ROWS - off16

    def pbody(g, carry):
        i16 = s * IMG_ROWS + g * 16 + iota16
        b16 = jnp.zeros((16,), I32)
        for m in range(1, BS):
            om = off16[m]
            b16 = b16 + jnp.where(i16 >= jnp.full((16,), om, I32), 1, 0)
        corr16 = plsc.load_gather(corr_v, [b16])
        st16 = jnp.minimum(i16 + corr16, GARBAGE_ROW)
        perm_v[pl.ds(s * IMG_ROWS + g * 16, 16)] = st16
        return carry
    lax.fori_loop(0, IMG_ROWS // 16, pbody, 0)
    pltpu.sync_copy(perm_v.at[pl.ds(s * IMG_ROWS, IMG_ROWS)],
                    perm_sh.at[pl.ds(s * IMG_ROWS, IMG_ROWS)])
    plsc.subcore_barrier()
    pltpu.sync_copy(perm_sh, perm_v)

    @pl.when(c == 0)
    def _():
        pltpu.async_copy(
            table_hbm.at[perm_v.at[pl.ds(s * IMG_ROWS, CH4)]],
            tokA, sem_g).wait()
        pltpu.sync_copy(tokA, fc_out.at[pl.ds(s * IMG_ROWS, CH4)])
        pltpu.async_copy(
            table_hbm.at[perm_v.at[pl.ds(s * IMG_ROWS + CH4, CH4)]],
            tokA, sem_g).wait()
        pltpu.sync_copy(tokA, fc_out.at[pl.ds(s * IMG_ROWS + CH4, CH4)])
        pltpu.async_copy(
            table_hbm.at[perm_v.at[pl.ds(s * IMG_ROWS + 2 * CH4, 16)]],
            tokA.at[pl.ds(0, 16)], sem_g).wait()
        pltpu.sync_copy(tokA.at[pl.ds(0, 16)],
                        fc_out.at[pl.ds(s * IMG_ROWS + 2 * CH4, 16)])

    toks = [tokA, tokB]
    sidxs = [sidxA, sidxB]
    s2s = [s2A, s2B]
    didxs = [didxA, didxB]

    def fire_idx(ch):
        r0 = wid * (EPT // RPC) + ch * CALLS4
        w = ch % 2
        return [pltpu.async_copy(edge_hbm.at[0, pl.ds(r0, CALLS4)],
                                 sidxs[w], sem_in),
                pltpu.async_copy(edge_hbm.at[1, pl.ds(r0, CALLS4)],
                                 didxs[w], sem_in)]

    def remap(ch):
        w = ch % 2
        sidx, s2 = sidxs[w], s2s[w]
        for j in range(CALLS4):
            for k in range(8):
                v16 = sidx[j, pl.ds(k * 16, 16)]
                s2[j, pl.ds(k * 16, 16)] = plsc.load_gather(perm_v, [v16])

    def fire_gathers(ch):
        w = ch % 2
        tok, s2 = toks[w], s2s[w]
        return [pltpu.async_copy(table_hbm.at[s2.at[j]],
                                 tok.at[pl.ds(j * RPC, RPC)], sem_g)
                for j in range(CALLS4)]

    pend_idx = fire_idx(0)
    for cp in pend_idx:
        cp.wait()
    remap(0)
    pend_g = fire_gathers(0)
    pend_idx = fire_idx(1)

    for ch in range(NCH4):
        tok, didx = toks[ch % 2], didxs[ch % 2]
        for cp in pend_g:
            cp.wait()
        adds = []
        for j in range(CALLS4):
            adds.append(pltpu.async_copy(
                tok.at[pl.ds(j * RPC, RPC)], sums_sh.at[didx.at[j]],
                sem_add, add=True))
            adds.append(pltpu.async_copy(
                ones_v, cnt_sh.at[didx.at[j]], sem_add, add=True))
        if ch + 1 < NCH4:
            for cp in pend_idx:
                cp.wait()
            remap(ch + 1)
            pend_g = fire_gathers(ch + 1)
        for cp in adds:
            cp.wait()
        if ch + 2 < NCH4:
            pend_idx = fire_idx(ch + 2)

    plsc.subcore_barrier()
    pltpu.sync_copy(sums_sh.at[pl.ds(s * IMG_ROWS, IMG_ROWS)],
                    sums_out.at[c, pl.ds(s * IMG_ROWS, IMG_ROWS)])
    pltpu.sync_copy(cnt_sh.at[pl.ds(s * IMG_ROWS, IMG_ROWS)],
                    cnt_out.at[c, pl.ds(s * IMG_ROWS, IMG_ROWS)])


_edge_accum = functools.partial(
    pl.kernel,
    out_type=(jax.ShapeDtypeStruct((NC, SEG_PAD, D), F32),
              jax.ShapeDtypeStruct((NC, SEG_PAD, CW), F32),
              jax.ShapeDtypeStruct((SEG_PAD, D), F32)),
    mesh=_MESH,
    compiler_params=_SC_PARAMS,
    scratch_types=[
        pltpu.VMEM((CH4, D), F32),
        pltpu.VMEM((CH4, D), F32),
        pltpu.VMEM((CALLS4, RPC), I32),
        pltpu.VMEM((CALLS4, RPC), I32),
        pltpu.VMEM((CALLS4, RPC), I32),
        pltpu.VMEM((CALLS4, RPC), I32),
        pltpu.VMEM((CALLS4, RPC), I32),
        pltpu.VMEM((CALLS4, RPC), I32),
        pltpu.VMEM((BS,), I32),
        pltpu.VMEM((BS,), I32),
        pltpu.VMEM((SEG_PAD,), I32),
        pltpu.VMEM((RPC, CW), F32),
        pltpu.VMEM((ZR, CW), F32),
        pltpu.VMEM_SHARED((SEG_PAD, D), F32),
        pltpu.VMEM_SHARED((SEG_PAD, CW), F32),
        pltpu.VMEM_SHARED((SEG_PAD,), I32),
        pltpu.SemaphoreType.DMA,
        pltpu.SemaphoreType.DMA,
        pltpu.SemaphoreType.DMA,
    ],
)(_edge_body)


_RB5 = SEG_PAD // 8


def _k5_body(f_ref, a_ref, c_ref, w_ref, out_ref):
    agg = (a_ref[0] + a_ref[1]) / jnp.maximum(
        c_ref[0, :, 0:1] + c_ref[1, :, 0:1], 1.0)
    t = f_ref[...] + agg
    o = jnp.dot(t, w_ref[...], preferred_element_type=F32)
    n = jnp.sqrt(jnp.sum(o * o, axis=1, keepdims=True)) + 1e-12
    out_ref[...] = o / n


def _head(f_compact, agg_sums, agg_cnts, W):
    return pl.pallas_call(
        _k5_body,
        grid=(SEG_PAD // _RB5,),
        in_specs=[
            pl.BlockSpec((_RB5, D), lambda i: (i, 0)),
            pl.BlockSpec((NC, _RB5, D), lambda i: (0, i, 0)),
            pl.BlockSpec((NC, _RB5, CW), lambda i: (0, i, 0)),
            pl.BlockSpec((D, D), lambda i: (0, 0)),
        ],
        out_specs=pl.BlockSpec((_RB5, D), lambda i: (i, 0)),
        out_shape=jax.ShapeDtypeStruct((SEG_PAD, D), F32),
    )(f_compact, agg_sums, agg_cnts, W)


def kernel(embeddings, sp_seg, edge_index, W):
    off = _offsets(sp_seg).reshape(BS)
    table = _pool(embeddings, sp_seg)
    edge3 = edge_index.reshape(2, E // RPC, RPC)
    agg_sums, agg_cnts, f_compact = _edge_accum(table, edge3, off)
    out = _head(f_compact, agg_sums, agg_cnts, W)
    return out[:NUM_SEG]
